# initial kernel scaffold (unmeasured)
import jax
import jax.numpy as jnp
from jax import lax
from jax.experimental import pallas as pl
from jax.experimental.pallas import tpu as pltpu

N_DEV = 32
M_BLK = 128
K_BLK = 128
N_OUT = 2048

FP8 = jnp.float8_e4m3fn


def kernel(x, w_mat, scale_x, scale_w):
    k_total, k_per = x.shape
    _, n = w_mat.shape
    assert k_per == K_BLK and n == N_OUT, (x.shape, w_mat.shape)

    def body(x_ref, w_ref, sx_ref, sw_ref, out_ref,
             x8, a2a, send_sems, recv_sems, local_sem):
        me = lax.axis_index("i")

        x8[...] = x_ref[...].astype(FP8)

        barrier = pltpu.get_barrier_semaphore()
        for k in range(1, N_DEV):
            peer = lax.rem(me + k, N_DEV)
            pl.semaphore_signal(
                barrier, inc=1,
                device_id=(peer,), device_id_type=pl.DeviceIdType.MESH,
            )
        pl.semaphore_wait(barrier, N_DEV - 1)

        lcopy = pltpu.make_async_copy(
            x8.at[pl.ds(me * M_BLK, M_BLK), :], a2a.at[me], local_sem,
        )
        lcopy.start()

        for k in range(1, N_DEV):
            dst = lax.rem(me + k, N_DEV)
            rdma = pltpu.make_async_remote_copy(
                src_ref=x8.at[pl.ds(dst * M_BLK, M_BLK), :],
                dst_ref=a2a.at[me],
                send_sem=send_sems.at[k],
                recv_sem=recv_sems.at[me],
                device_id=(dst,),
                device_id_type=pl.DeviceIdType.MESH,
            )
            rdma.start()

        for k in range(1, N_DEV):
            src = lax.rem(me + N_DEV - k, N_DEV)
            rwait = pltpu.make_async_remote_copy(
                src_ref=x8.at[pl.ds(0, M_BLK), :],
                dst_ref=a2a.at[src],
                send_sem=send_sems.at[k],
                recv_sem=recv_sems.at[src],
                device_id=(0,),
                device_id_type=pl.DeviceIdType.MESH,
            )
            rwait.wait_recv()
        lcopy.wait()

        scale = sx_ref[0] * sw_ref[0]
        acc = jnp.zeros((M_BLK, N_OUT), jnp.float32)
        for j in range(N_DEV):
            a = a2a[j].astype(jnp.bfloat16)
            wj = w_ref[pl.ds(j * K_BLK, K_BLK), :].astype(jnp.bfloat16)
            acc = acc + jnp.dot(a, wj, preferred_element_type=jnp.float32)
        y = acc * scale
        out_ref[...] = y * jax.nn.sigmoid(y)

        for k in range(1, N_DEV):
            swait = pltpu.make_async_remote_copy(
                src_ref=x8.at[pl.ds(0, M_BLK), :],
                dst_ref=a2a.at[0],
                send_sem=send_sems.at[k],
                recv_sem=recv_sems.at[0],
                device_id=(0,),
                device_id_type=pl.DeviceIdType.MESH,
            )
            swait.wait_send()

    return pl.pallas_call(
        body,
        out_shape=jax.ShapeDtypeStruct((M_BLK, N_OUT), jnp.float32),
        in_specs=[
            pl.BlockSpec(memory_space=pltpu.VMEM),
            pl.BlockSpec(memory_space=pltpu.VMEM),
            pl.BlockSpec(memory_space=pltpu.SMEM),
            pl.BlockSpec(memory_space=pltpu.SMEM),
        ],
        out_specs=pl.BlockSpec(memory_space=pltpu.VMEM),
        scratch_shapes=[
            pltpu.VMEM((k_total, K_BLK), FP8),
            pltpu.VMEM((N_DEV, M_BLK, K_BLK), FP8),
            pltpu.SemaphoreType.DMA((N_DEV,)),
            pltpu.SemaphoreType.DMA((N_DEV,)),
            pltpu.SemaphoreType.DMA,
        ],
        compiler_params=pltpu.CompilerParams(collective_id=0),
    )(x, w_mat, scale_x, scale_w)


# baseline (device time: 35375 ns/iter reference)
import jax
import jax.numpy as jnp
from jax import lax
from jax.experimental import pallas as pl
from jax.experimental.pallas import tpu as pltpu

N_DEV = 32
M_BLK = 128
K_BLK = 128
N_OUT = 2048

FP8 = jnp.float8_e4m3fn


def kernel(x, w_mat, scale_x, scale_w):
    k_total, k_per = x.shape
    _, n = w_mat.shape
    assert k_per == K_BLK and n == N_OUT, (x.shape, w_mat.shape)

    def body(x_ref, w_ref, sx_ref, sw_ref, out_ref,
             x8, a2a, send_sems, recv_sems, local_sem):
        me = lax.axis_index("i")

        x8[...] = x_ref[...].astype(FP8)

        barrier = pltpu.get_barrier_semaphore()
        for k in range(1, N_DEV):
            peer = lax.rem(me + k, N_DEV)
            pl.semaphore_signal(
                barrier, inc=1,
                device_id=(peer,), device_id_type=pl.DeviceIdType.MESH,
            )
        pl.semaphore_wait(barrier, N_DEV - 1)

        lcopy = pltpu.make_async_copy(
            x8.at[pl.ds(me * M_BLK, M_BLK), :], a2a.at[me], local_sem,
        )
        lcopy.start()

        for k in range(1, N_DEV):
            dst = lax.rem(me + k, N_DEV)
            rdma = pltpu.make_async_remote_copy(
                src_ref=x8.at[pl.ds(dst * M_BLK, M_BLK), :],
                dst_ref=a2a.at[me],
                send_sem=send_sems.at[k],
                recv_sem=recv_sems.at[me],
                device_id=(dst,),
                device_id_type=pl.DeviceIdType.MESH,
            )
            rdma.start()

        for k in range(1, N_DEV):
            src = lax.rem(me + N_DEV - k, N_DEV)
            rwait = pltpu.make_async_remote_copy(
                src_ref=x8.at[pl.ds(0, M_BLK), :],
                dst_ref=a2a.at[src],
                send_sem=send_sems.at[k],
                recv_sem=recv_sems.at[src],
                device_id=(0,),
                device_id_type=pl.DeviceIdType.MESH,
            )
            rwait.wait_recv()
        lcopy.wait()

        scale = sx_ref[0] * sw_ref[0]
        acc = jnp.zeros((M_BLK, N_OUT), jnp.float32)
        for j in range(N_DEV):
            a = a2a[j].astype(jnp.bfloat16)
            wj = w_ref[pl.ds(j * K_BLK, K_BLK), :].astype(jnp.bfloat16)
            acc = acc + jnp.dot(a, wj, preferred_element_type=jnp.float32)
        y = acc * scale
        out_ref[...] = y * jax.nn.sigmoid(y)

        for k in range(1, N_DEV):
            swait = pltpu.make_async_remote_copy(
                src_ref=x8.at[pl.ds(0, M_BLK), :],
                dst_ref=a2a.at[0],
                send_sem=send_sems.at[k],
                recv_sem=recv_sems.at[0],
                device_id=(0,),
                device_id_type=pl.DeviceIdType.MESH,
            )
            swait.wait_send()

    return pl.pallas_call(
        body,
        out_shape=jax.ShapeDtypeStruct((M_BLK, N_OUT), jnp.float32),
        in_specs=[
            pl.BlockSpec(memory_space=pltpu.VMEM),
            pl.BlockSpec(memory_space=pltpu.VMEM),
            pl.BlockSpec(memory_space=pltpu.SMEM),
            pl.BlockSpec(memory_space=pltpu.SMEM),
        ],
        out_specs=pl.BlockSpec(memory_space=pltpu.VMEM),
        scratch_shapes=[
            pltpu.VMEM((k_total, K_BLK), FP8),
            pltpu.VMEM((N_DEV, M_BLK, K_BLK), FP8),
            pltpu.SemaphoreType.DMA((N_DEV,)),
            pltpu.SemaphoreType.DMA((N_DEV,)),
            pltpu.SemaphoreType.DMA,
        ],
        compiler_params=pltpu.CompilerParams(
            collective_id=0,
            vmem_limit_bytes=100 * 1024 * 1024,
        ),
    )(x, w_mat, scale_x, scale_w)


# device time: 31980 ns/iter; 1.1062x vs baseline; 1.1062x over previous
import jax
import jax.numpy as jnp
from jax import lax
from jax.experimental import pallas as pl
from jax.experimental.pallas import tpu as pltpu

N_DEV = 32
M_BLK = 128
K_BLK = 128
N_OUT = 2048

FP8 = jnp.float8_e4m3fn


def kernel(x, w_mat, scale_x, scale_w):
    k_total, k_per = x.shape
    _, n = w_mat.shape
    assert k_per == K_BLK and n == N_OUT, (x.shape, w_mat.shape)

    def body(x_ref, w_ref, sx_ref, sw_ref, out_ref,
             x8, xg, w8, send_sems, recv_sems, local_sem):
        me = lax.axis_index("i")

        x8[...] = x_ref[...].astype(FP8)

        barrier = pltpu.get_barrier_semaphore()
        for k in range(1, N_DEV):
            peer = lax.rem(me + k, N_DEV)
            pl.semaphore_signal(
                barrier, inc=1,
                device_id=(peer,), device_id_type=pl.DeviceIdType.MESH,
            )
        pl.semaphore_wait(barrier, N_DEV - 1)

        for k in range(1, N_DEV):
            dst = lax.rem(me + k, N_DEV)
            rdma = pltpu.make_async_remote_copy(
                src_ref=x8.at[pl.ds(dst * M_BLK, M_BLK), :],
                dst_ref=xg.at[:, pl.ds(me * K_BLK, K_BLK)],
                send_sem=send_sems.at[k],
                recv_sem=recv_sems.at[me],
                device_id=(dst,),
                device_id_type=pl.DeviceIdType.MESH,
            )
            rdma.start()

        lcopy = pltpu.make_async_copy(
            x8.at[pl.ds(me * M_BLK, M_BLK), :],
            xg.at[:, pl.ds(me * K_BLK, K_BLK)],
            local_sem,
        )
        lcopy.start()

        w8[...] = w_ref[...].astype(FP8)

        for k in range(1, N_DEV):
            src = lax.rem(me + N_DEV - k, N_DEV)
            rwait = pltpu.make_async_remote_copy(
                src_ref=x8.at[pl.ds(0, M_BLK), :],
                dst_ref=xg.at[:, pl.ds(src * K_BLK, K_BLK)],
                send_sem=send_sems.at[k],
                recv_sem=recv_sems.at[src],
                device_id=(0,),
                device_id_type=pl.DeviceIdType.MESH,
            )
            rwait.wait_recv()
        lcopy.wait()

        scale = sx_ref[0] * sw_ref[0]
        acc = jnp.dot(xg[...], w8[...], preferred_element_type=jnp.float32)
        y = acc * scale
        out_ref[...] = y * jax.nn.sigmoid(y)

        for k in range(1, N_DEV):
            swait = pltpu.make_async_remote_copy(
                src_ref=x8.at[pl.ds(0, M_BLK), :],
                dst_ref=xg.at[:, pl.ds(0, K_BLK)],
                send_sem=send_sems.at[k],
                recv_sem=recv_sems.at[0],
                device_id=(0,),
                device_id_type=pl.DeviceIdType.MESH,
            )
            swait.wait_send()

    return pl.pallas_call(
        body,
        out_shape=jax.ShapeDtypeStruct((M_BLK, N_OUT), jnp.float32),
        in_specs=[
            pl.BlockSpec(memory_space=pltpu.VMEM),
            pl.BlockSpec(memory_space=pltpu.VMEM),
            pl.BlockSpec(memory_space=pltpu.SMEM),
            pl.BlockSpec(memory_space=pltpu.SMEM),
        ],
        out_specs=pl.BlockSpec(memory_space=pltpu.VMEM),
        scratch_shapes=[
            pltpu.VMEM((k_total, K_BLK), FP8),
            pltpu.VMEM((M_BLK, k_total), FP8),
            pltpu.VMEM((k_total, N_OUT), FP8),
            pltpu.SemaphoreType.DMA((N_DEV,)),
            pltpu.SemaphoreType.DMA((N_DEV,)),
            pltpu.SemaphoreType.DMA,
        ],
        compiler_params=pltpu.CompilerParams(
            collective_id=0,
            vmem_limit_bytes=100 * 1024 * 1024,
        ),
    )(x, w_mat, scale_x, scale_w)


# device time: 22297 ns/iter; 1.5865x vs baseline; 1.4343x over previous
import jax
import jax.numpy as jnp
from jax import lax
from jax.experimental import pallas as pl
from jax.experimental.pallas import tpu as pltpu

N_DEV = 32
M_BLK = 128
K_BLK = 128
N_OUT = 2048
N_WCHUNK = 4

FP8 = jnp.float8_e4m3fn


def kernel(x, w_mat, scale_x, scale_w):
    k_total, k_per = x.shape
    _, n = w_mat.shape
    assert k_per == K_BLK and n == N_OUT, (x.shape, w_mat.shape)
    wc = k_total // N_WCHUNK

    def body(x_ref, w_ref, sx_ref, sw_ref, out_ref,
             x8, xg, w32, w8, send_sems, recv_sems, w_sems, local_sem,
             ready_sems):
        me = lax.axis_index("i")

        for k in range(1, N_DEV):
            peer = lax.rem(me + k, N_DEV)
            pl.semaphore_signal(
                ready_sems.at[me], inc=1,
                device_id=(peer,), device_id_type=pl.DeviceIdType.MESH,
            )

        barrier = pltpu.get_barrier_semaphore()
        pl.semaphore_signal(barrier, inc=1)
        pl.semaphore_wait(barrier, 1)

        x8[...] = x_ref[...].astype(FP8)

        for c in range(N_WCHUNK):
            pltpu.make_async_copy(
                w_ref.at[pl.ds(c * wc, wc), :],
                w32.at[pl.ds(c * wc, wc), :],
                w_sems.at[c],
            ).start()

        for k in range(1, N_DEV):
            dst = lax.rem(me + k, N_DEV)
            pl.semaphore_wait(ready_sems.at[dst], 1)
            pltpu.make_async_remote_copy(
                src_ref=x8.at[pl.ds(dst * M_BLK, M_BLK), :],
                dst_ref=xg.at[:, pl.ds(me * K_BLK, K_BLK)],
                send_sem=send_sems.at[k],
                recv_sem=recv_sems.at[me],
                device_id=(dst,),
                device_id_type=pl.DeviceIdType.MESH,
            ).start()

        lcopy = pltpu.make_async_copy(
            x8.at[pl.ds(me * M_BLK, M_BLK), :],
            xg.at[:, pl.ds(me * K_BLK, K_BLK)],
            local_sem,
        )
        lcopy.start()

        for c in range(N_WCHUNK):
            pltpu.make_async_copy(
                w_ref.at[pl.ds(c * wc, wc), :],
                w32.at[pl.ds(c * wc, wc), :],
                w_sems.at[c],
            ).wait()
            w8[pl.ds(c * wc, wc), :] = w32[pl.ds(c * wc, wc), :].astype(FP8)

        for k in range(1, N_DEV):
            src = lax.rem(me + N_DEV - k, N_DEV)
            pltpu.make_async_remote_copy(
                src_ref=x8.at[pl.ds(0, M_BLK), :],
                dst_ref=xg.at[:, pl.ds(src * K_BLK, K_BLK)],
                send_sem=send_sems.at[k],
                recv_sem=recv_sems.at[src],
                device_id=(0,),
                device_id_type=pl.DeviceIdType.MESH,
            ).wait_recv()
        lcopy.wait()

        scale = sx_ref[0] * sw_ref[0]
        acc = jnp.dot(xg[...], w8[...], preferred_element_type=jnp.float32)
        y = acc * scale
        out_ref[...] = y * jax.nn.sigmoid(y)

        for k in range(1, N_DEV):
            pltpu.make_async_remote_copy(
                src_ref=x8.at[pl.ds(0, M_BLK), :],
                dst_ref=xg.at[:, pl.ds(0, K_BLK)],
                send_sem=send_sems.at[k],
                recv_sem=recv_sems.at[0],
                device_id=(0,),
                device_id_type=pl.DeviceIdType.MESH,
            ).wait_send()

    return pl.pallas_call(
        body,
        out_shape=jax.ShapeDtypeStruct((M_BLK, N_OUT), jnp.float32),
        in_specs=[
            pl.BlockSpec(memory_space=pltpu.VMEM),
            pl.BlockSpec(memory_space=pl.ANY),
            pl.BlockSpec(memory_space=pltpu.SMEM),
            pl.BlockSpec(memory_space=pltpu.SMEM),
        ],
        out_specs=pl.BlockSpec(memory_space=pltpu.VMEM),
        scratch_shapes=[
            pltpu.VMEM((k_total, K_BLK), FP8),
            pltpu.VMEM((M_BLK, k_total), FP8),
            pltpu.VMEM((k_total, N_OUT), jnp.float32),
            pltpu.VMEM((k_total, N_OUT), FP8),
            pltpu.SemaphoreType.DMA((N_DEV,)),
            pltpu.SemaphoreType.DMA((N_DEV,)),
            pltpu.SemaphoreType.DMA((N_WCHUNK,)),
            pltpu.SemaphoreType.DMA,
            pltpu.SemaphoreType.REGULAR((N_DEV,)),
        ],
        compiler_params=pltpu.CompilerParams(
            collective_id=0,
            vmem_limit_bytes=100 * 1024 * 1024,
        ),
    )(x, w_mat, scale_x, scale_w)


# device time: 21398 ns/iter; 1.6532x vs baseline; 1.0420x over previous
import jax
import jax.numpy as jnp
from jax import lax
from jax.experimental import pallas as pl
from jax.experimental.pallas import tpu as pltpu

N_DEV = 32
M_BLK = 128
K_BLK = 128
N_OUT = 2048
N_PHASE = 4
GRP = 8
K_CHUNK = GRP * K_BLK

FP8 = jnp.float8_e4m3fn


def kernel(x, w_mat, scale_x, scale_w):
    k_total, k_per = x.shape
    _, n = w_mat.shape
    assert k_per == K_BLK and n == N_OUT, (x.shape, w_mat.shape)
    wc = k_total // N_PHASE

    def body(x_ref, w_ref, sx_ref, sw_ref, out_ref,
             x8, xg, w32, w8, acc_ref, send_sems, recv_sems, w_sems,
             local_sem):
        me = lax.axis_index("i")
        me_g = me // GRP
        me_l = lax.rem(me, GRP)

        x8[...] = x_ref[...].astype(FP8)

        barrier = pltpu.get_barrier_semaphore()
        for k in range(1, N_DEV):
            peer = lax.rem(me + k, N_DEV)
            pl.semaphore_signal(
                barrier, inc=1,
                device_id=(peer,), device_id_type=pl.DeviceIdType.MESH,
            )

        for c in range(N_PHASE):
            pltpu.make_async_copy(
                w_ref.at[pl.ds(c * wc, wc), :],
                w32.at[pl.ds(c * wc, wc), :],
                w_sems.at[c],
            ).start()

        pl.semaphore_wait(barrier, N_DEV - 1)

        for r in range(1, N_DEV):
            a, b = r // GRP, r % GRP
            dst = (lax.rem(me_g + a, N_PHASE)) * GRP + lax.rem(me_l + b, GRP)
            pltpu.make_async_remote_copy(
                src_ref=x8.at[pl.ds(dst * M_BLK, M_BLK), :],
                dst_ref=xg.at[:, pl.ds(me * K_BLK, K_BLK)],
                send_sem=send_sems.at[r],
                recv_sem=recv_sems.at[me],
                device_id=(dst,),
                device_id_type=pl.DeviceIdType.MESH,
            ).start()

        lcopy = pltpu.make_async_copy(
            x8.at[pl.ds(me * M_BLK, M_BLK), :],
            xg.at[:, pl.ds(me * K_BLK, K_BLK)],
            local_sem,
        )
        lcopy.start()

        for c in range(N_PHASE):
            pltpu.make_async_copy(
                w_ref.at[pl.ds(c * wc, wc), :],
                w32.at[pl.ds(c * wc, wc), :],
                w_sems.at[c],
            ).wait()
            w8[pl.ds(c * wc, wc), :] = w32[pl.ds(c * wc, wc), :].astype(FP8)
        lcopy.wait()

        scale = sx_ref[0] * sw_ref[0]
        acc_ref[...] = jnp.zeros((M_BLK, N_OUT), jnp.float32)
        for a in range(N_PHASE):
            c = lax.rem(me_g + N_PHASE - a, N_PHASE)
            for b in range(GRP):
                if a == 0 and b == 0:
                    continue
                src = c * GRP + lax.rem(me_l + GRP - b, GRP)
                pltpu.make_async_remote_copy(
                    src_ref=x8.at[pl.ds(0, M_BLK), :],
                    dst_ref=xg.at[:, pl.ds(src * K_BLK, K_BLK)],
                    send_sem=send_sems.at[1],
                    recv_sem=recv_sems.at[src],
                    device_id=(0,),
                    device_id_type=pl.DeviceIdType.MESH,
                ).wait_recv()
            ck = c * K_CHUNK
            acc_ref[...] += jnp.dot(
                xg[:, pl.ds(ck, K_CHUNK)],
                w8[pl.ds(ck, K_CHUNK), :],
                preferred_element_type=jnp.float32,
            )

        y = acc_ref[...] * scale
        out_ref[...] = y * jax.nn.sigmoid(y)

        for r in range(1, N_DEV):
            pltpu.make_async_remote_copy(
                src_ref=x8.at[pl.ds(0, M_BLK), :],
                dst_ref=xg.at[:, pl.ds(0, K_BLK)],
                send_sem=send_sems.at[r],
                recv_sem=recv_sems.at[0],
                device_id=(0,),
                device_id_type=pl.DeviceIdType.MESH,
            ).wait_send()

    return pl.pallas_call(
        body,
        out_shape=jax.ShapeDtypeStruct((M_BLK, N_OUT), jnp.float32),
        in_specs=[
            pl.BlockSpec(memory_space=pltpu.VMEM),
            pl.BlockSpec(memory_space=pl.ANY),
            pl.BlockSpec(memory_space=pltpu.SMEM),
            pl.BlockSpec(memory_space=pltpu.SMEM),
        ],
        out_specs=pl.BlockSpec(memory_space=pltpu.VMEM),
        scratch_shapes=[
            pltpu.VMEM((k_total, K_BLK), FP8),
            pltpu.VMEM((M_BLK, k_total), FP8),
            pltpu.VMEM((k_total, N_OUT), jnp.float32),
            pltpu.VMEM((k_total, N_OUT), FP8),
            pltpu.VMEM((M_BLK, N_OUT), jnp.float32),
            pltpu.SemaphoreType.DMA((N_DEV,)),
            pltpu.SemaphoreType.DMA((N_DEV,)),
            pltpu.SemaphoreType.DMA((N_PHASE,)),
            pltpu.SemaphoreType.DMA,
        ],
        compiler_params=pltpu.CompilerParams(
            collective_id=0,
            vmem_limit_bytes=100 * 1024 * 1024,
        ),
    )(x, w_mat, scale_x, scale_w)
